# baseline (speedup 1.0000x reference)
import jax
import jax.numpy as jnp
from jax import lax
from jax.experimental import pallas as pl
from jax.experimental.pallas import tpu as pltpu
from jax.experimental.pallas import tpu_sc as plsc

NUM_CORES = 2
NUM_SUBCORES = 16
LANES = 16
NW = NUM_CORES * NUM_SUBCORES

ROWS_PER_CHUNK = 8


def _make_sc_kernel(B, D, E, S):
    assert B % NW == 0 and D % ROWS_PER_CHUNK == 0 and E % 2 == 0
    bpw = B // NW
    n_chunks = D // ROWS_PER_CHUNK
    n_groups = bpw // LANES
    mesh = plsc.VectorSubcoreMesh(core_axis_name="c", subcore_axis_name="s")

    def body(pairs_hbm, ewt_hbm, swt_hbm, out_hbm,
             eidh_v, epar_v, sids_v, ev_v, sv_v, acc_v, out_v,
             spbufs, swt_sp, sem_sp0, sem_sp1, sem_ev, sem_sw):
        sid = lax.axis_index("s")
        wid = sid * NUM_CORES + lax.axis_index("c")
        base = wid * bpw
        sems = [sem_sp0, sem_sp1]

        @pl.when(sid == 0)
        def _():
            pltpu.async_copy(swt_hbm, swt_sp, sem_sw)

        pltpu.sync_copy(pairs_hbm.at[0, pl.ds(base, bpw)], eidh_v)
        pltpu.sync_copy(pairs_hbm.at[1, pl.ds(base, bpw)], sids_v)
        for c in range(n_groups):
            sl = pl.ds(c * LANES, LANES)
            ei = jnp.minimum(jnp.maximum(eidh_v[sl], 0), E - 1)
            eidh_v[sl] = ei >> 1
            epar_v[sl] = (ei & 1) << 4
            sids_v[sl] = jnp.minimum(jnp.maximum(sids_v[sl], 0), S - 1)

        def stripe_args(c):
            return (ewt_hbm.at[pl.ds(c * ROWS_PER_CHUNK, ROWS_PER_CHUNK)],
                    spbufs.at[c % 2], sems[c % 2])

        def stage(c):
            @pl.when(sid == 0)
            def _():
                pltpu.async_copy(*stripe_args(c))

        def wait_stage(c):
            @pl.when(sid == 0)
            def _():
                pltpu.make_async_copy(*stripe_args(c)).wait()

        stage(0)

        @pl.when(sid == 0)
        def _():
            pltpu.make_async_copy(swt_hbm, swt_sp, sem_sw).wait()

        for c in range(n_chunks):
            wait_stage(c)
            plsc.subcore_barrier()
            if c + 1 < n_chunks:
                stage(c + 1)

            cps = []
            for j in range(ROWS_PER_CHUNK):
                cps.append(pltpu.async_copy(
                    spbufs.at[c % 2].at[j].at[eidh_v],
                    ev_v.at[j], sem_ev))
                cps.append(pltpu.async_copy(
                    swt_sp.at[c * ROWS_PER_CHUNK + j].at[sids_v],
                    sv_v.at[j], sem_ev))
            for cp in cps:
                cp.wait()

            def g_body(g, carry, c=c):
                i0 = g * LANES
                par = epar_v[pl.ds(i0, LANES)]
                acc = acc_v[pl.ds(i0, LANES)] if c else jnp.zeros((LANES,), jnp.float32)
                for j in range(ROWS_PER_CHUNK):
                    pairbits = ev_v[j, pl.ds(i0, LANES)]
                    ev = plsc.bitcast(
                        (lax.shift_right_logical(pairbits, par) << 16), jnp.float32)
                    acc = acc + ev * sv_v[j, pl.ds(i0, LANES)]
                if c + 1 < n_chunks:
                    acc_v[pl.ds(i0, LANES)] = acc
                else:
                    out_v[pl.ds(i0, LANES)] = 1.0 / (1.0 + jnp.exp(-acc))
                return carry

            lax.fori_loop(0, n_groups, g_body, 0)

        pltpu.sync_copy(out_v, out_hbm.at[pl.ds(base, bpw)])

    return pl.kernel(
        body,
        out_type=jax.ShapeDtypeStruct((B,), jnp.float32),
        mesh=mesh,
        compiler_params=pltpu.CompilerParams(
            needs_layout_passes=False, use_tc_tiling_on_sc=False),
        scratch_types=[
            pltpu.VMEM((bpw,), jnp.int32),
            pltpu.VMEM((bpw,), jnp.int32),
            pltpu.VMEM((bpw,), jnp.int32),
            pltpu.VMEM((ROWS_PER_CHUNK, bpw), jnp.int32),
            pltpu.VMEM((ROWS_PER_CHUNK, bpw), jnp.float32),
            pltpu.VMEM((bpw,), jnp.float32),
            pltpu.VMEM((bpw,), jnp.float32),
            pltpu.VMEM_SHARED((2, ROWS_PER_CHUNK, E // 2), jnp.int32),
            pltpu.VMEM_SHARED((D, S), jnp.float32),
            pltpu.SemaphoreType.DMA,
            pltpu.SemaphoreType.DMA,
            pltpu.SemaphoreType.DMA,
            pltpu.SemaphoreType.DMA,
        ],
    )


def kernel(pairs, exercise_w, skill_w):
    B = pairs.shape[0]
    E, D = exercise_w.shape
    S = skill_w.shape[0]
    sc = _make_sc_kernel(B, D, E, S)
    ewt32 = jax.lax.bitcast_convert_type(
        exercise_w.T.astype(jnp.bfloat16).reshape(D, E // 2, 2), jnp.int32)
    return sc(pairs.T, ewt32, skill_w.T)


# speedup vs baseline: 5.1279x; 5.1279x over previous
import jax
import jax.numpy as jnp
from jax import lax
from jax.experimental import pallas as pl
from jax.experimental.pallas import tpu as pltpu
from jax.experimental.pallas import tpu_sc as plsc

NUM_CORES = 2
NUM_SUBCORES = 16
LANES = 16
NW = NUM_CORES * NUM_SUBCORES

IDX_CHUNK = 128


def _make_sc_kernel(B, D, E, S):
    assert B % NW == 0
    bpw = B // NW
    n_chunks = bpw // IDX_CHUNK
    n_groups = bpw // LANES
    mesh = plsc.VectorSubcoreMesh(core_axis_name="c", subcore_axis_name="s")

    def body(pairs_hbm, ew_hbm, sw_hbm, out_hbm,
             pairs_v, eidx_v, sidx_v, erows_v, stab_v, out_v, sem):
        wid = lax.axis_index("s") * NUM_CORES + lax.axis_index("c")
        base = wid * bpw
        lane = lax.iota(jnp.int32, LANES)

        stab_cp = pltpu.async_copy(sw_hbm, stab_v, sem)

        pltpu.sync_copy(pairs_hbm.at[pl.ds(base * 2, bpw * 2)], pairs_v)

        for c in range(bpw // LANES):
            src = c * 2 * LANES + lane * 2
            ei = plsc.load_gather(pairs_v, [src])
            si = plsc.load_gather(pairs_v, [src + 1])
            ei = jnp.minimum(jnp.maximum(ei, 0), E - 1)
            si = jnp.minimum(jnp.maximum(si, 0), S - 1)
            row, off = divmod(c * LANES, IDX_CHUNK)
            eidx_v[row, pl.ds(off, LANES)] = ei
            sidx_v[pl.ds(c * LANES, LANES)] = si

        copies = []
        for j in range(n_chunks):
            dst = pl.ds(j * IDX_CHUNK, IDX_CHUNK)
            copies.append(pltpu.async_copy(ew_hbm.at[eidx_v.at[j]], erows_v.at[dst], sem))
        for cp in copies:
            cp.wait()
        stab_cp.wait()

        def g_body(g, carry):
            r = g * LANES + lane
            sid = sidx_v[pl.ds(g * LANES, LANES)]
            acc = jnp.zeros((LANES,), jnp.float32)
            for d in range(D):
                dv = jnp.full((LANES,), d, jnp.int32)
                ev = plsc.load_gather(erows_v, [r, dv])
                sv = plsc.load_gather(stab_v, [sid, dv])
                acc = acc + ev * sv
            out_v[pl.ds(g * LANES, LANES)] = 1.0 / (1.0 + jnp.exp(-acc))
            return carry

        lax.fori_loop(0, n_groups, g_body, 0)
        pltpu.sync_copy(out_v, out_hbm.at[pl.ds(base, bpw)])

    return pl.kernel(
        body,
        out_type=jax.ShapeDtypeStruct((B,), jnp.float32),
        mesh=mesh,
        compiler_params=pltpu.CompilerParams(
            needs_layout_passes=False, use_tc_tiling_on_sc=False),
        scratch_types=[
            pltpu.VMEM((2 * bpw,), jnp.int32),
            pltpu.VMEM((n_chunks, IDX_CHUNK), jnp.int32),
            pltpu.VMEM((bpw,), jnp.int32),
            pltpu.VMEM((bpw, D), jnp.float32),
            pltpu.VMEM((S, D), jnp.float32),
            pltpu.VMEM((bpw,), jnp.float32),
            pltpu.SemaphoreType.DMA,
        ],
    )


def kernel(pairs, exercise_w, skill_w):
    B = pairs.shape[0]
    E, D = exercise_w.shape
    S = skill_w.shape[0]
    sc = _make_sc_kernel(B, D, E, S)
    return sc(pairs.reshape(-1), exercise_w, skill_w)


# speedup vs baseline: 5.3405x; 1.0415x over previous
import jax
import jax.numpy as jnp
from jax import lax
from jax.experimental import pallas as pl
from jax.experimental.pallas import tpu as pltpu
from jax.experimental.pallas import tpu_sc as plsc

NUM_CORES = 2
NUM_SUBCORES = 16
LANES = 16
NW = NUM_CORES * NUM_SUBCORES

IDX_CHUNK = 128


def _make_sc_kernel(B, D, E, S):
    assert B % NW == 0
    bpw = B // NW
    n_chunks = bpw // IDX_CHUNK
    n_groups = bpw // LANES
    mesh = plsc.VectorSubcoreMesh(core_axis_name="c", subcore_axis_name="s")

    def body(pairs_hbm, ew_hbm, sw_hbm, out_hbm,
             eflat_v, eidx_v, sidx_v, erows_v, stab_v, out_v, sem):
        wid = lax.axis_index("s") * NUM_CORES + lax.axis_index("c")
        base = wid * bpw
        lane = lax.iota(jnp.int32, LANES)

        stab_cp = pltpu.async_copy(sw_hbm, stab_v, sem)

        pltpu.sync_copy(pairs_hbm.at[0, pl.ds(base, bpw)], eflat_v)
        pltpu.sync_copy(pairs_hbm.at[1, pl.ds(base, bpw)], sidx_v)

        for c in range(bpw // LANES):
            sl = pl.ds(c * LANES, LANES)
            ei = jnp.minimum(jnp.maximum(eflat_v[sl], 0), E - 1)
            row, off = divmod(c * LANES, IDX_CHUNK)
            eidx_v[row, pl.ds(off, LANES)] = ei
            sidx_v[sl] = jnp.minimum(jnp.maximum(sidx_v[sl], 0), S - 1)

        copies = []
        for j in range(n_chunks):
            dst = pl.ds(j * IDX_CHUNK, IDX_CHUNK)
            copies.append(pltpu.async_copy(ew_hbm.at[eidx_v.at[j]], erows_v.at[dst], sem))
        for cp in copies:
            cp.wait()
        stab_cp.wait()

        def g_body(g, carry):
            r = g * LANES + lane
            sid = sidx_v[pl.ds(g * LANES, LANES)]
            acc = jnp.zeros((LANES,), jnp.float32)
            for d in range(D):
                dv = jnp.full((LANES,), d, jnp.int32)
                ev = plsc.load_gather(erows_v, [r, dv])
                sv = plsc.load_gather(stab_v, [sid, dv])
                acc = acc + ev * sv
            out_v[pl.ds(g * LANES, LANES)] = 1.0 / (1.0 + jnp.exp(-acc))
            return carry

        lax.fori_loop(0, n_groups, g_body, 0)
        pltpu.sync_copy(out_v, out_hbm.at[pl.ds(base, bpw)])

    return pl.kernel(
        body,
        out_type=jax.ShapeDtypeStruct((B,), jnp.float32),
        mesh=mesh,
        compiler_params=pltpu.CompilerParams(
            needs_layout_passes=False, use_tc_tiling_on_sc=False),
        scratch_types=[
            pltpu.VMEM((bpw,), jnp.int32),
            pltpu.VMEM((n_chunks, IDX_CHUNK), jnp.int32),
            pltpu.VMEM((bpw,), jnp.int32),
            pltpu.VMEM((bpw, D), jnp.float32),
            pltpu.VMEM((S, D), jnp.float32),
            pltpu.VMEM((bpw,), jnp.float32),
            pltpu.SemaphoreType.DMA,
        ],
    )


def kernel(pairs, exercise_w, skill_w):
    B = pairs.shape[0]
    E, D = exercise_w.shape
    S = skill_w.shape[0]
    sc = _make_sc_kernel(B, D, E, S)
    return sc(pairs.T, exercise_w, skill_w)
